# baseline (speedup 1.0000x reference)
import functools

import jax
import jax.numpy as jnp
from jax import lax
from jax.experimental import pallas as pl
from jax.experimental.pallas import tpu as pltpu
from jax.experimental.pallas import tpu_sc as plsc

_PTS = 64
_IMP = 64
_EPS = 1e-5
_CH = 64
_L = 16


def _sc_call(n, half, opacities, depths, b1_f, b2_f):
    info = plsc.get_sparse_core_info()
    nc, ns = info.num_cores, info.num_subcores
    nw = nc * ns
    nh = n // 2
    base0 = half * nh
    rpw = nh // nw
    nch = rpw // _CH
    npair = nch // 2
    groups = _CH // _L
    csz = _CH * 128
    cdfg = _PTS * _L
    histg = (_IMP + 2) * _L

    mesh = plsc.VectorSubcoreMesh(core_axis_name="c", subcore_axis_name="s")

    @functools.partial(
        pl.kernel,
        out_type=[jax.ShapeDtypeStruct((nh, _PTS), jnp.float32),
                  jax.ShapeDtypeStruct((nh, _PTS), jnp.float32),
                  jax.ShapeDtypeStruct((nh, _PTS), jnp.float32),
                  jax.ShapeDtypeStruct((nh, 4), jnp.float32)],
        mesh=mesh,
        scratch_types=[
            pltpu.VMEM((2 * _CH, _PTS), jnp.float32),
            pltpu.VMEM((2 * _CH, _PTS), jnp.float32),
            pltpu.VMEM((2 * csz,), jnp.float32),
            pltpu.VMEM((2 * csz,), jnp.float32),
            pltpu.VMEM((_CH, _PTS), jnp.float32),
            pltpu.VMEM((_CH, _PTS), jnp.float32),
            pltpu.VMEM((_CH, _PTS), jnp.float32),
            pltpu.VMEM((_CH, 4), jnp.float32),
            pltpu.VMEM((4 * cdfg,), jnp.float32),
            pltpu.VMEM((4 * histg,), jnp.int32),
            pltpu.SemaphoreType.DMA,
            pltpu.SemaphoreType.DMA,
            pltpu.SemaphoreType.DMA,
        ],
        compiler_params=pltpu.CompilerParams(needs_layout_passes=False),
    )
    def body(op_h, dep_h, b1_h, b2_h, yx_h, yy_h, yz_h, aq_h,
             op_vm, dep_vm, b1_vm, b2_vm, yx_vm, yy_vm, yz_vm, aq_vm,
             cdf_vm, hist_vm, si0, si1, so):
        wid = lax.axis_index("s") * nc + lax.axis_index("c")
        iota = lax.iota(jnp.int32, _L)
        zf = jnp.zeros((_L,), jnp.float32)
        onef = jnp.ones((_L,), jnp.float32)
        onei = jnp.ones((_L,), jnp.int32)
        zi = jnp.zeros((_L,), jnp.int32)
        si = (si0, si1)

        def in_copies(ch, slot):
            rows = pl.ds(base0 + wid * rpw + ch * _CH, _CH)
            off = pl.ds((base0 + wid * rpw) * 128 + ch * csz, csz)
            v = pl.ds(slot * _CH, _CH)
            b = pl.ds(slot * csz, csz)
            s = si[slot]
            return (
                pltpu.make_async_copy(op_h.at[rows], op_vm.at[v], s),
                pltpu.make_async_copy(dep_h.at[rows], dep_vm.at[v], s),
                pltpu.make_async_copy(b1_h.at[off], b1_vm.at[b], s),
                pltpu.make_async_copy(b2_h.at[off], b2_vm.at[b], s),
            )

        def out_copies(ch):
            rows = pl.ds(wid * rpw + ch * _CH, _CH)
            return (
                pltpu.make_async_copy(yx_vm, yx_h.at[rows], so),
                pltpu.make_async_copy(yy_vm, yy_h.at[rows], so),
                pltpu.make_async_copy(yz_vm, yz_h.at[rows], so),
                pltpu.make_async_copy(aq_vm, aq_h.at[rows], so),
            )

        def compute_group(g, slot):
            row = slot * _CH + g * _L + iota
            grow = g * _L + iota
            ray_b = slot * csz + grow * 128
            coff = g * cdfg
            hoff = g * histg

            @plsc.parallel_loop(0, _IMP + 2, unroll=8)
            def hzero(v):
                hist_vm[pl.ds(hoff + v * _L, _L)] = zi

            ob = ray_b + 64
            o0 = plsc.load_gather(b2_vm, [ob])
            o1 = plsc.load_gather(b2_vm, [ob + 1])
            o2 = plsc.load_gather(b2_vm, [ob + 2])
            e0 = plsc.load_gather(b2_vm, [ob + 3])
            e1 = plsc.load_gather(b2_vm, [ob + 4])
            e2 = plsc.load_gather(b2_vm, [ob + 5])

            @plsc.parallel_loop(0, _PTS, unroll=8,
                               carry=(onef, zf, zf, zf, zf))
            def march(k, carry):
                trans, cdf, a0, a1, a2 = carry
                opk = plsc.load_gather(op_vm, [row, zi + k])
                w = opk * trans
                trans = trans * (1.0 - opk)
                cdf = cdf + (w + _EPS)
                cdf_vm[pl.ds(coff + k * _L, _L)] = cdf
                vb = ray_b + k
                a0 = a0 + w * plsc.load_gather(b1_vm, [vb])
                a1 = a1 + w * plsc.load_gather(b1_vm, [vb + 64])
                a2 = a2 + w * plsc.load_gather(b2_vm, [vb])
                return trans, cdf, a0, a1, a2

            _, ctot, a0, a1, a2 = march
            acc_o = jnp.clip(ctot - _PTS * _EPS, 0.0, 1.0)
            plsc.store_scatter(aq_vm, [grow, zi], a0)
            plsc.store_scatter(aq_vm, [grow, zi + 1], a1)
            plsc.store_scatter(aq_vm, [grow, zi + 2], a2)
            plsc.store_scatter(aq_vm, [grow, zi + 3], acc_o)

            scale = jnp.float32(_IMP) / ctot

            @plsc.parallel_loop(0, _PTS, unroll=8)
            def bink(k):
                ck = cdf_vm[pl.ds(coff + k * _L, _L)]
                x = ck * scale
                xi = x.astype(jnp.int32)
                xi = xi + (xi.astype(jnp.float32) < x).astype(jnp.int32)
                m = jnp.minimum(xi, _IMP + 1)
                plsc.addupdate_scatter(hist_vm, [hoff + m * _L + iota], onei)

            c_first = cdf_vm[pl.ds(coff, _L)]
            d_first = plsc.load_gather(dep_vm, [row, zi])
            d_last = plsc.load_gather(dep_vm, [row, zi + (_PTS - 1)])

            @plsc.parallel_loop(1, _IMP + 1, unroll=8,
                               carry=(d_first, hist_vm[pl.ds(hoff, _L)]))
            def sample(j, carry):
                f_prev, cnt = carry
                cnt = cnt + hist_vm[pl.ds(hoff + j * _L, _L)]
                i = jnp.clip(cnt, 1, _PTS - 1)
                g0 = coff + (i - 1) * _L + iota
                c0 = plsc.load_gather(cdf_vm, [g0])
                c1 = plsc.load_gather(cdf_vm, [g0 + _L])
                d0 = plsc.load_gather(dep_vm, [row, i - 1])
                d1 = plsc.load_gather(dep_vm, [row, i])
                uj = lax.convert_element_type(j, jnp.float32) * (1.0 / _IMP)
                u = uj * ctot
                f = d0 + ((u - c0) / (c1 - c0)) * (d1 - d0)
                f = jnp.where(u < c_first, d_first, f)
                f = jnp.where(u >= ctot, d_last, f)
                mid = 0.5 * (f_prev + f)
                jm = zi + (j - 1)
                plsc.store_scatter(yx_vm, [grow, jm], o0 + mid * e0)
                plsc.store_scatter(yy_vm, [grow, jm], o1 + mid * e1)
                plsc.store_scatter(yz_vm, [grow, jm], o2 + mid * e2)
                return f, cnt

            del sample

        def compute_chunk(slot):
            @plsc.parallel_loop(0, groups, unroll=2)
            def grp(g):
                compute_group(g, slot)

        def half(ch, slot, has_prev_out):
            for cp in in_copies(ch, slot):
                cp.wait()
            @pl.when(has_prev_out)
            def _():
                for cp in out_copies(ch - 1):
                    cp.wait()

            compute_chunk(slot)
            for cp in out_copies(ch):
                cp.start()
            @pl.when(ch + 2 < nch)
            def _():
                for cp in in_copies(ch + 2, slot):
                    cp.start()

        def pair_body(p, _):
            ch0 = p * 2
            half(ch0, 0, p > 0)
            half(ch0 + 1, 1, ch0 + 1 > 0)
            return 0

        for cp in in_copies(0, 0):
            cp.start()
        for cp in in_copies(1, 1):
            cp.start()
        lax.fori_loop(0, npair, pair_body, 0)
        for cp in out_copies(nch - 1):
            cp.wait()

    return body(opacities, depths, b1_f, b2_f)


def _assemble(yx, yy, yz, accq):
    nh = yx.shape[0]
    coords = jnp.stack([yx, yy, yz], axis=-1)
    return jnp.concatenate(
        [accq[:, None, 0:3],
         jnp.broadcast_to(accq[:, 3:4, None], (nh, 1, 3)),
         coords], axis=1)


def kernel(opacities, values, depths, origins, dirs):
    n = opacities.shape[0]
    b1 = (jnp.pad(values[:, :, 0], ((0, 0), (0, 64)))
          + jnp.pad(values[:, :, 1], ((0, 0), (64, 0))))
    b2 = (jnp.pad(values[:, :, 2], ((0, 0), (0, 64)))
          + jnp.pad(origins, ((0, 0), (64, 61)))
          + jnp.pad(dirs, ((0, 0), (67, 58))))
    b1f = b1.reshape(-1)
    b2f = b2.reshape(-1)
    out0 = _assemble(*_sc_call(n, 0, opacities, depths, b1f, b2f))
    out1 = _assemble(*_sc_call(n, 1, opacities, depths, b1f, b2f))
    return jnp.concatenate([out0, out1], axis=0)


# speedup vs baseline: 1.0057x; 1.0057x over previous
import functools

import jax
import jax.numpy as jnp
from jax import lax
from jax.experimental import pallas as pl
from jax.experimental.pallas import tpu as pltpu
from jax.experimental.pallas import tpu_sc as plsc

_PTS = 64
_IMP = 64
_EPS = 1e-5
_CH = 64
_L = 16


def _sc_call(n, opacities, depths, b1_f, b2_f):
    info = plsc.get_sparse_core_info()
    nc, ns = info.num_cores, info.num_subcores
    nw = nc * ns
    rpw = n // nw
    nch = rpw // _CH
    npair = nch // 2
    groups = _CH // _L
    csz = _CH * 128
    cdfg = _PTS * _L
    histg = (_IMP + 2) * _L

    mesh = plsc.VectorSubcoreMesh(core_axis_name="c", subcore_axis_name="s")

    @functools.partial(
        pl.kernel,
        out_type=[jax.ShapeDtypeStruct((n, _PTS), jnp.float32),
                  jax.ShapeDtypeStruct((n, _PTS), jnp.float32),
                  jax.ShapeDtypeStruct((n, _PTS), jnp.float32),
                  jax.ShapeDtypeStruct((n, 4), jnp.float32)],
        mesh=mesh,
        scratch_types=[
            pltpu.VMEM((2 * _CH, _PTS), jnp.float32),
            pltpu.VMEM((2 * _CH, _PTS), jnp.float32),
            pltpu.VMEM((2 * csz,), jnp.float32),
            pltpu.VMEM((2 * csz,), jnp.float32),
            pltpu.VMEM((_CH, _PTS), jnp.float32),
            pltpu.VMEM((_CH, _PTS), jnp.float32),
            pltpu.VMEM((_CH, _PTS), jnp.float32),
            pltpu.VMEM((_CH, 4), jnp.float32),
            pltpu.VMEM((4 * cdfg,), jnp.float32),
            pltpu.VMEM((4 * histg,), jnp.int32),
            pltpu.SemaphoreType.DMA,
            pltpu.SemaphoreType.DMA,
            pltpu.SemaphoreType.DMA,
        ],
        compiler_params=pltpu.CompilerParams(needs_layout_passes=False),
    )
    def body(op_h, dep_h, b1_h, b2_h, yx_h, yy_h, yz_h, aq_h,
             op_vm, dep_vm, b1_vm, b2_vm, yx_vm, yy_vm, yz_vm, aq_vm,
             cdf_vm, hist_vm, si0, si1, so):
        wid = lax.axis_index("s") * nc + lax.axis_index("c")
        iota = lax.iota(jnp.int32, _L)
        zf = jnp.zeros((_L,), jnp.float32)
        onef = jnp.ones((_L,), jnp.float32)
        onei = jnp.ones((_L,), jnp.int32)
        zi = jnp.zeros((_L,), jnp.int32)
        si = (si0, si1)

        def in_copies(ch, slot):
            rows = pl.ds(wid * rpw + ch * _CH, _CH)
            off = pl.ds(wid * rpw * 128 + ch * csz, csz)
            v = pl.ds(slot * _CH, _CH)
            b = pl.ds(slot * csz, csz)
            s = si[slot]
            return (
                pltpu.make_async_copy(op_h.at[rows], op_vm.at[v], s),
                pltpu.make_async_copy(dep_h.at[rows], dep_vm.at[v], s),
                pltpu.make_async_copy(b1_h.at[off], b1_vm.at[b], s),
                pltpu.make_async_copy(b2_h.at[off], b2_vm.at[b], s),
            )

        def out_copies(ch):
            rows = pl.ds(wid * rpw + ch * _CH, _CH)
            return (
                pltpu.make_async_copy(yx_vm, yx_h.at[rows], so),
                pltpu.make_async_copy(yy_vm, yy_h.at[rows], so),
                pltpu.make_async_copy(yz_vm, yz_h.at[rows], so),
                pltpu.make_async_copy(aq_vm, aq_h.at[rows], so),
            )

        def compute_group(g, slot):
            row = slot * _CH + g * _L + iota
            grow = g * _L + iota
            ray_b = slot * csz + grow * 128
            coff = g * cdfg
            hoff = g * histg

            @plsc.parallel_loop(0, _IMP + 2, unroll=8)
            def hzero(v):
                hist_vm[pl.ds(hoff + v * _L, _L)] = zi

            ob = ray_b + 64
            o0 = plsc.load_gather(b2_vm, [ob])
            o1 = plsc.load_gather(b2_vm, [ob + 1])
            o2 = plsc.load_gather(b2_vm, [ob + 2])
            e0 = plsc.load_gather(b2_vm, [ob + 3])
            e1 = plsc.load_gather(b2_vm, [ob + 4])
            e2 = plsc.load_gather(b2_vm, [ob + 5])

            @plsc.parallel_loop(0, _PTS, unroll=16,
                               carry=(onef, zf, zf, zf, zf))
            def march(k, carry):
                trans, cdf, a0, a1, a2 = carry
                opk = plsc.load_gather(op_vm, [row, zi + k])
                w = opk * trans
                trans = trans * (1.0 - opk)
                cdf = cdf + (w + _EPS)
                cdf_vm[pl.ds(coff + k * _L, _L)] = cdf
                vb = ray_b + k
                a0 = a0 + w * plsc.load_gather(b1_vm, [vb])
                a1 = a1 + w * plsc.load_gather(b1_vm, [vb + 64])
                a2 = a2 + w * plsc.load_gather(b2_vm, [vb])
                return trans, cdf, a0, a1, a2

            _, ctot, a0, a1, a2 = march
            acc_o = jnp.clip(ctot - _PTS * _EPS, 0.0, 1.0)
            plsc.store_scatter(aq_vm, [grow, zi], a0)
            plsc.store_scatter(aq_vm, [grow, zi + 1], a1)
            plsc.store_scatter(aq_vm, [grow, zi + 2], a2)
            plsc.store_scatter(aq_vm, [grow, zi + 3], acc_o)

            scale = jnp.float32(_IMP) / ctot

            @plsc.parallel_loop(0, _PTS, unroll=8)
            def bink(k):
                ck = cdf_vm[pl.ds(coff + k * _L, _L)]
                x = ck * scale
                xi = x.astype(jnp.int32)
                xi = xi + (xi.astype(jnp.float32) < x).astype(jnp.int32)
                m = jnp.minimum(xi, _IMP + 1)
                plsc.addupdate_scatter(hist_vm, [hoff + m * _L + iota], onei)

            c_first = cdf_vm[pl.ds(coff, _L)]
            d_first = plsc.load_gather(dep_vm, [row, zi])
            d_last = plsc.load_gather(dep_vm, [row, zi + (_PTS - 1)])

            @plsc.parallel_loop(1, _IMP + 1, unroll=16,
                               carry=(d_first, hist_vm[pl.ds(hoff, _L)]))
            def sample(j, carry):
                f_prev, cnt = carry
                cnt = cnt + hist_vm[pl.ds(hoff + j * _L, _L)]
                i = jnp.clip(cnt, 1, _PTS - 1)
                g0 = coff + (i - 1) * _L + iota
                c0 = plsc.load_gather(cdf_vm, [g0])
                c1 = plsc.load_gather(cdf_vm, [g0 + _L])
                d0 = plsc.load_gather(dep_vm, [row, i - 1])
                d1 = plsc.load_gather(dep_vm, [row, i])
                uj = lax.convert_element_type(j, jnp.float32) * (1.0 / _IMP)
                u = uj * ctot
                f = d0 + ((u - c0) / (c1 - c0)) * (d1 - d0)
                f = jnp.where(u < c_first, d_first, f)
                f = jnp.where(u >= ctot, d_last, f)
                mid = 0.5 * (f_prev + f)
                jm = zi + (j - 1)
                plsc.store_scatter(yx_vm, [grow, jm], o0 + mid * e0)
                plsc.store_scatter(yy_vm, [grow, jm], o1 + mid * e1)
                plsc.store_scatter(yz_vm, [grow, jm], o2 + mid * e2)
                return f, cnt

            del sample

        def compute_chunk(slot):
            @plsc.parallel_loop(0, groups, unroll=2)
            def grp(g):
                compute_group(g, slot)

        def half(ch, slot, has_prev_out):
            for cp in in_copies(ch, slot):
                cp.wait()
            @pl.when(has_prev_out)
            def _():
                for cp in out_copies(ch - 1):
                    cp.wait()

            compute_chunk(slot)
            for cp in out_copies(ch):
                cp.start()
            @pl.when(ch + 2 < nch)
            def _():
                for cp in in_copies(ch + 2, slot):
                    cp.start()

        def pair_body(p, _):
            ch0 = p * 2
            half(ch0, 0, p > 0)
            half(ch0 + 1, 1, ch0 + 1 > 0)
            return 0

        for cp in in_copies(0, 0):
            cp.start()
        for cp in in_copies(1, 1):
            cp.start()
        lax.fori_loop(0, npair, pair_body, 0)
        for cp in out_copies(nch - 1):
            cp.wait()

    return body(opacities, depths, b1_f, b2_f)


def kernel(opacities, values, depths, origins, dirs):
    n = opacities.shape[0]
    b1 = (jnp.pad(values[:, :, 0], ((0, 0), (0, 64)))
          + jnp.pad(values[:, :, 1], ((0, 0), (64, 0))))
    b2 = (jnp.pad(values[:, :, 2], ((0, 0), (0, 64)))
          + jnp.pad(origins, ((0, 0), (64, 61)))
          + jnp.pad(dirs, ((0, 0), (67, 58))))
    yx, yy, yz, accq = _sc_call(n, opacities, depths,
                                b1.reshape(-1), b2.reshape(-1))
    coords = jnp.stack([yx, yy, yz], axis=-1)
    return jnp.concatenate(
        [accq[:, None, 0:3],
         jnp.broadcast_to(accq[:, 3:4, None], (n, 1, 3)),
         coords], axis=1)


# speedup vs baseline: 1.0088x; 1.0031x over previous
import functools

import jax
import jax.numpy as jnp
from jax import lax
from jax.experimental import pallas as pl
from jax.experimental.pallas import tpu as pltpu
from jax.experimental.pallas import tpu_sc as plsc

_PTS = 64
_IMP = 64
_EPS = 1e-5
_CH = 64
_L = 16


def _sc_call(n, opacities, depths, b1_f, b2_f):
    info = plsc.get_sparse_core_info()
    nc, ns = info.num_cores, info.num_subcores
    nw = nc * ns
    rpw = n // nw
    nch = rpw // _CH
    npair = nch // 2
    groups = _CH // _L
    csz = _CH * 128
    cdfg = _PTS * _L
    histg = (_IMP + 2) * _L

    mesh = plsc.VectorSubcoreMesh(core_axis_name="c", subcore_axis_name="s")

    @functools.partial(
        pl.kernel,
        out_type=[jax.ShapeDtypeStruct((n, _PTS), jnp.float32),
                  jax.ShapeDtypeStruct((n, _PTS), jnp.float32),
                  jax.ShapeDtypeStruct((n, _PTS), jnp.float32),
                  jax.ShapeDtypeStruct((n, 4), jnp.float32)],
        mesh=mesh,
        scratch_types=[
            pltpu.VMEM((2 * _CH, _PTS), jnp.float32),
            pltpu.VMEM((2 * _CH, _PTS), jnp.float32),
            pltpu.VMEM((2 * _CH, 128), jnp.float32),
            pltpu.VMEM((2 * _CH, 128), jnp.float32),
            pltpu.VMEM((_CH, _PTS), jnp.float32),
            pltpu.VMEM((_CH, _PTS), jnp.float32),
            pltpu.VMEM((_CH, _PTS), jnp.float32),
            pltpu.VMEM((_CH, 4), jnp.float32),
            pltpu.VMEM((4 * cdfg,), jnp.float32),
            pltpu.VMEM((4 * histg,), jnp.int32),
            pltpu.SemaphoreType.DMA,
            pltpu.SemaphoreType.DMA,
            pltpu.SemaphoreType.DMA,
        ],
        compiler_params=pltpu.CompilerParams(needs_layout_passes=False),
    )
    def body(op_h, dep_h, b1_h, b2_h, yx_h, yy_h, yz_h, aq_h,
             op_vm, dep_vm, b1_vm, b2_vm, yx_vm, yy_vm, yz_vm, aq_vm,
             cdf_vm, hist_vm, si0, si1, so):
        wid = lax.axis_index("s") * nc + lax.axis_index("c")
        iota = lax.iota(jnp.int32, _L)
        zf = jnp.zeros((_L,), jnp.float32)
        onef = jnp.ones((_L,), jnp.float32)
        onei = jnp.ones((_L,), jnp.int32)
        zi = jnp.zeros((_L,), jnp.int32)
        si = (si0, si1)

        def in_copies(ch, slot):
            rows = pl.ds(wid * rpw + ch * _CH, _CH)
            v = pl.ds(slot * _CH, _CH)
            s = si[slot]
            return (
                pltpu.make_async_copy(op_h.at[rows], op_vm.at[v], s),
                pltpu.make_async_copy(dep_h.at[rows], dep_vm.at[v], s),
                pltpu.make_async_copy(b1_h.at[rows], b1_vm.at[v], s),
                pltpu.make_async_copy(b2_h.at[rows], b2_vm.at[v], s),
            )

        def out_copies(ch):
            rows = pl.ds(wid * rpw + ch * _CH, _CH)
            return (
                pltpu.make_async_copy(yx_vm, yx_h.at[rows], so),
                pltpu.make_async_copy(yy_vm, yy_h.at[rows], so),
                pltpu.make_async_copy(yz_vm, yz_h.at[rows], so),
                pltpu.make_async_copy(aq_vm, aq_h.at[rows], so),
            )

        def compute_group(g, slot):
            row = slot * _CH + g * _L + iota
            grow = g * _L + iota
            coff = g * cdfg
            hoff = g * histg

            @plsc.parallel_loop(0, _IMP + 2, unroll=8)
            def hzero(v):
                hist_vm[pl.ds(hoff + v * _L, _L)] = zi

            ob = zi + 64
            o0 = plsc.load_gather(b2_vm, [row, ob])
            o1 = plsc.load_gather(b2_vm, [row, ob + 1])
            o2 = plsc.load_gather(b2_vm, [row, ob + 2])
            e0 = plsc.load_gather(b2_vm, [row, ob + 3])
            e1 = plsc.load_gather(b2_vm, [row, ob + 4])
            e2 = plsc.load_gather(b2_vm, [row, ob + 5])

            @plsc.parallel_loop(0, _PTS, unroll=8,
                               carry=(onef, zf, zf, zf, zf))
            def march(k, carry):
                trans, cdf, a0, a1, a2 = carry
                opk = plsc.load_gather(op_vm, [row, zi + k])
                w = opk * trans
                trans = trans * (1.0 - opk)
                cdf = cdf + (w + _EPS)
                cdf_vm[pl.ds(coff + k * _L, _L)] = cdf
                kv = zi + k
                a0 = a0 + w * plsc.load_gather(b1_vm, [row, kv])
                a1 = a1 + w * plsc.load_gather(b1_vm, [row, kv + 64])
                a2 = a2 + w * plsc.load_gather(b2_vm, [row, kv])
                return trans, cdf, a0, a1, a2

            _, ctot, a0, a1, a2 = march
            acc_o = jnp.clip(ctot - _PTS * _EPS, 0.0, 1.0)
            plsc.store_scatter(aq_vm, [grow, zi], a0)
            plsc.store_scatter(aq_vm, [grow, zi + 1], a1)
            plsc.store_scatter(aq_vm, [grow, zi + 2], a2)
            plsc.store_scatter(aq_vm, [grow, zi + 3], acc_o)

            scale = jnp.float32(_IMP) / ctot

            @plsc.parallel_loop(0, _PTS, unroll=8)
            def bink(k):
                ck = cdf_vm[pl.ds(coff + k * _L, _L)]
                x = ck * scale
                xi = x.astype(jnp.int32)
                xi = xi + (xi.astype(jnp.float32) < x).astype(jnp.int32)
                m = jnp.minimum(xi, _IMP + 1)
                plsc.addupdate_scatter(hist_vm, [hoff + m * _L + iota], onei)

            c_first = cdf_vm[pl.ds(coff, _L)]
            d_first = plsc.load_gather(dep_vm, [row, zi])
            d_last = plsc.load_gather(dep_vm, [row, zi + (_PTS - 1)])

            @plsc.parallel_loop(1, _IMP + 1, unroll=8,
                               carry=(d_first, hist_vm[pl.ds(hoff, _L)]))
            def sample(j, carry):
                f_prev, cnt = carry
                cnt = cnt + hist_vm[pl.ds(hoff + j * _L, _L)]
                i = jnp.clip(cnt, 1, _PTS - 1)
                g0 = coff + (i - 1) * _L + iota
                c0 = plsc.load_gather(cdf_vm, [g0])
                c1 = plsc.load_gather(cdf_vm, [g0 + _L])
                d0 = plsc.load_gather(dep_vm, [row, i - 1])
                d1 = plsc.load_gather(dep_vm, [row, i])
                uj = lax.convert_element_type(j, jnp.float32) * (1.0 / _IMP)
                u = uj * ctot
                f = d0 + ((u - c0) / (c1 - c0)) * (d1 - d0)
                f = jnp.where(u < c_first, d_first, f)
                f = jnp.where(u >= ctot, d_last, f)
                mid = 0.5 * (f_prev + f)
                jm = zi + (j - 1)
                plsc.store_scatter(yx_vm, [grow, jm], o0 + mid * e0)
                plsc.store_scatter(yy_vm, [grow, jm], o1 + mid * e1)
                plsc.store_scatter(yz_vm, [grow, jm], o2 + mid * e2)
                return f, cnt

            del sample

        def compute_chunk(slot):
            @plsc.parallel_loop(0, groups, unroll=2)
            def grp(g):
                compute_group(g, slot)

        def half(ch, slot, has_prev_out):
            for cp in in_copies(ch, slot):
                cp.wait()
            @pl.when(has_prev_out)
            def _():
                for cp in out_copies(ch - 1):
                    cp.wait()

            compute_chunk(slot)
            for cp in out_copies(ch):
                cp.start()
            @pl.when(ch + 2 < nch)
            def _():
                for cp in in_copies(ch + 2, slot):
                    cp.start()

        def pair_body(p, _):
            ch0 = p * 2
            half(ch0, 0, p > 0)
            half(ch0 + 1, 1, ch0 + 1 > 0)
            return 0

        for cp in in_copies(0, 0):
            cp.start()
        for cp in in_copies(1, 1):
            cp.start()
        lax.fori_loop(0, npair, pair_body, 0)
        for cp in out_copies(nch - 1):
            cp.wait()

    return body(opacities, depths, b1_f, b2_f)


def kernel(opacities, values, depths, origins, dirs):
    n = opacities.shape[0]
    b1 = (jnp.pad(values[:, :, 0], ((0, 0), (0, 64)))
          + jnp.pad(values[:, :, 1], ((0, 0), (64, 0))))
    b2 = (jnp.pad(values[:, :, 2], ((0, 0), (0, 64)))
          + jnp.pad(origins, ((0, 0), (64, 61)))
          + jnp.pad(dirs, ((0, 0), (67, 58))))
    yx, yy, yz, accq = _sc_call(n, opacities, depths, b1, b2)
    coords = jnp.stack([yx, yy, yz], axis=-1)
    return jnp.concatenate(
        [accq[:, None, 0:3],
         jnp.broadcast_to(accq[:, 3:4, None], (n, 1, 3)),
         coords], axis=1)
